# Initial kernel scaffold; baseline (speedup 1.0000x reference)
import jax
import jax.numpy as jnp
from jax.experimental import pallas as pl


def kernel(x, edge_index, W1, b1, gamma, beta, W2, b2):
    raise NotImplementedError("write your pallas kernel here")


# speedup vs baseline: 2.5353x; 2.5353x over previous
import functools

import jax
import jax.numpy as jnp
from jax import lax
from jax.experimental import pallas as pl
from jax.experimental.pallas import tpu as pltpu
from jax.experimental.pallas import tpu_sc as plsc

N = 10000
D = 128
E = 320000
NC = 2
NS = 16
NW = NC * NS
C = 128
CHUNKS = 80
EPAD = NW * CHUNKS * C
ROWS = 10112
DUMP = ROWS - 1
RPT = ROWS // NS
SLABS = NW * CHUNKS
EPS = 1e-5

_mesh = plsc.VectorSubcoreMesh(core_axis_name="c", subcore_axis_name="s")


@functools.partial(
    pl.kernel,
    out_type=jax.ShapeDtypeStruct((NC * ROWS,), jnp.float32),
    mesh=_mesh,
    scratch_types=[
        pltpu.VMEM((CHUNKS,), jnp.int32),
        pltpu.VMEM((CHUNKS, C), jnp.int32),
        pltpu.VMEM((C,), jnp.float32),
        pltpu.VMEM((RPT,), jnp.float32),
        pltpu.SemaphoreType.DMA,
        pltpu.VMEM_SHARED((ROWS,), jnp.float32),
    ],
)
def _sc_hist(dst_hbm, degp_hbm, rowidx_v, idx_v, ones_v, str_v, sem, acc_sh):
    c = lax.axis_index("c")
    s = lax.axis_index("s")
    w = c * NS + s

    for g in range(C // 16):
        ones_v[pl.ds(g * 16, 16)] = jnp.full((16,), 1.0, jnp.float32)
    k = 0
    while k < RPT:
        k0 = min(k, RPT - 16)
        str_v[pl.ds(k0, 16)] = jnp.zeros((16,), jnp.float32)
        k += 16

    pltpu.sync_copy(str_v, acc_sh.at[pl.ds(s * RPT, RPT)])

    k = 0
    while k < CHUNKS:
        k0 = min(k, CHUNKS - 16)
        rowidx_v[pl.ds(k0, 16)] = (w * CHUNKS + k0
                                   + jax.lax.iota(jnp.int32, 16))
        k += 16
    pltpu.async_copy(dst_hbm.at[rowidx_v], idx_v, sem).wait()
    plsc.subcore_barrier()

    def step(j, carry):
        pltpu.sync_copy(ones_v, acc_sh.at[idx_v.at[j]], add=True)
        return carry

    lax.fori_loop(0, CHUNKS, step, 0)
    plsc.subcore_barrier()

    pltpu.sync_copy(acc_sh.at[pl.ds(s * RPT, RPT)], str_v)
    pltpu.sync_copy(str_v, degp_hbm.at[pl.ds(c * ROWS + s * RPT, RPT)])


def _tc1_body(x_ref, w1_ref, degp_ref, hs_ref, dinv_ref):
    deg = (degp_ref[0, :N] + degp_ref[1, :N] + 1.0).reshape(N, 1)
    dinv = lax.rsqrt(deg)
    dinv_ref[...] = dinv
    h = jnp.dot(x_ref[...], w1_ref[...], preferred_element_type=jnp.float32)
    hs_ref[...] = h * dinv


def _tc2_body(part_ref, hs1_ref, dinv_ref, g_ref, b1_ref, bt_ref, w2_ref,
              hs2_ref):
    dcol = dinv_ref[...]
    a = dcol * (part_ref[:N, :] + hs1_ref[...]) + b1_ref[...]
    mu = jnp.mean(a, axis=0, keepdims=True)
    var = jnp.mean(a * a, axis=0, keepdims=True) - mu * mu
    hbn = g_ref[...] * (a - mu) * lax.rsqrt(var + EPS) + bt_ref[...]
    hr = jnp.maximum(hbn, 0.0)
    h2 = jnp.dot(hr, w2_ref[...], preferred_element_type=jnp.float32)
    hs2_ref[...] = h2 * dcol


def _tc3_body(part_ref, hs2_ref, dinv_ref, b2_ref, out_ref):
    dcol = dinv_ref[...]
    out_ref[...] = dcol * (part_ref[:N, :] + hs2_ref[...]) + b2_ref[...]


_full = jax.ShapeDtypeStruct((N, D), jnp.float32)
_tc1 = pl.pallas_call(
    _tc1_body,
    out_shape=(_full, jax.ShapeDtypeStruct((N, 1), jnp.float32)),
)
_tc2 = pl.pallas_call(_tc2_body, out_shape=_full)
_tc3 = pl.pallas_call(_tc3_body, out_shape=_full)


@jax.jit
def kernel(x, edge_index, W1, b1, gamma, beta, W2, b2):
    src = edge_index[0]
    dst = edge_index[1]
    pad = EPAD - E
    srcf = jnp.concatenate([src, jnp.zeros((pad,), jnp.int32)])
    dstf = jnp.concatenate([dst, jnp.full((pad,), DUMP, jnp.int32)])
    drows = dstf.reshape(SLABS, C)

    degp = _sc_hist(drows).reshape(NC, ROWS)
    hs1, dinv = _tc1(x, W1, degp)

    p1 = jnp.zeros((ROWS, D), jnp.float32).at[dstf].add(hs1[srcf])
    hs2 = _tc2(p1, hs1, dinv, gamma.reshape(1, D), b1.reshape(1, D),
               beta.reshape(1, D), W2)
    p2 = jnp.zeros((ROWS, D), jnp.float32).at[dstf].add(hs2[srcf])
    return _tc3(p2, hs2, dinv, b2.reshape(1, D))
